# baseline (speedup 1.0000x reference)
import functools

import jax
import jax.numpy as jnp
from jax import lax
from jax.experimental import pallas as pl
from jax.experimental.pallas import tpu as pltpu
from jax.experimental.pallas import tpu_sc as plsc

N = 10000
NP = 10016
E = 64000
EP = E + 64
D = 768
H1 = 8

_G = 96
_NC_MAX = (E + _G - 1) // _G + 1
_NS = 312
_NRB = 336

_SC_PARAMS = pltpu.CompilerParams(use_tc_tiling_on_sc=False)
_MESH = plsc.VectorSubcoreMesh(core_axis_name="c", subcore_axis_name="s")


def _lane_bcast(vec, lane):
    idx = jnp.full((16, 1), lane, jnp.int32)
    return lax.gather(
        vec, idx,
        lax.GatherDimensionNumbers(offset_dims=(), collapsed_slice_dims=(0,),
                                   start_index_map=(0,)),
        slice_sizes=(1,), mode=lax.GatherScatterMode.PROMISE_IN_BOUNDS)


def _prep1_body(w1_ref, as_ref, ad_ref, out_ref):
    h = pl.program_id(0)
    ohr = (lax.broadcasted_iota(jnp.int32, (1, 8), 1) == h).astype(jnp.float32)
    a_s = ohr @ as_ref[...]
    a_d = ohr @ ad_ref[...]
    us = lax.dot_general(w1_ref[...], a_s, (((1,), (1,)), ((), ())))
    ud = lax.dot_general(w1_ref[...], a_d, (((1,), (1,)), ((), ())))
    row = lax.broadcasted_iota(jnp.int32, (1, 32), 1)
    ohs = (row == h).astype(jnp.float32)
    ohd = (row == (h + 16)).astype(jnp.float32)

    @pl.when(h == 0)
    def _():
        out_ref[...] = jnp.zeros_like(out_ref)

    out_ref[...] += us @ ohs + ud @ ohd


def _prep1(W1, a1s, a1d):
    return pl.pallas_call(
        _prep1_body,
        grid=(H1,),
        in_specs=[pl.BlockSpec((768, 768), lambda h: (0, h)),
                  pl.BlockSpec((8, 768), lambda h: (0, 0)),
                  pl.BlockSpec((8, 768), lambda h: (0, 0))],
        out_specs=pl.BlockSpec((768, 32), lambda h: (0, 0)),
        out_shape=jax.ShapeDtypeStruct((768, 32), jnp.float32),
    )(W1, a1s, a1d)


def _prep2_body(w2_ref, a2s_ref, a2d_ref, fcw_ref, clsw_ref, fcb_ref, clsb_ref,
                u2c_ref, wc_ref, bc_ref):
    u2s = lax.dot_general(w2_ref[...], a2s_ref[...], (((1,), (1,)), ((), ())))
    u2d = lax.dot_general(w2_ref[...], a2d_ref[...], (((1,), (1,)), ((), ())))
    row = lax.broadcasted_iota(jnp.int32, (1, 32), 1)
    oh0 = (row == 0).astype(jnp.float32)
    oh16 = (row == 16).astype(jnp.float32)
    u2c_ref[...] = u2s @ oh0 + u2d @ oh16
    wc = fcw_ref[...] @ clsw_ref[...]
    wc_ref[...] = wc
    bc_ref[...] = fcb_ref[...] @ clsw_ref[...] + clsb_ref[...]


def _prep2(W2, a2s, a2d, fc_W, cls_W, fc_b, cls_b):
    return pl.pallas_call(
        _prep2_body,
        out_shape=[jax.ShapeDtypeStruct((6144, 32), jnp.float32),
                   jax.ShapeDtypeStruct((1536, 2), jnp.float32),
                   jax.ShapeDtypeStruct((1, 2), jnp.float32)],
    )(W2, a2s, a2d, fc_W, cls_W, fc_b[None, :], cls_b[None, :])


def _asad1_body(x_ref, u_ref, as_ref, ad_ref):
    y = x_ref[...] @ u_ref[...]
    as_ref[...] = y[:, :16]
    ad_ref[...] = y[:, 16:32]


def _asad1(x, Ucat1):
    return pl.pallas_call(
        _asad1_body,
        grid=(5,),
        in_specs=[pl.BlockSpec((2000, 768), lambda r: (r, 0)),
                  pl.BlockSpec((768, 32), lambda r: (0, 0))],
        out_specs=[pl.BlockSpec((2000, 16), lambda r: (r, 0)),
                   pl.BlockSpec((2000, 16), lambda r: (r, 0))],
        out_shape=[jax.ShapeDtypeStruct((N, 16), jnp.float32),
                   jax.ShapeDtypeStruct((N, 16), jnp.float32)],
    )(x, Ucat1)


def _edge1_body(x_h, as_h, ad_h, src_h, dst_h, meta_h, zrow_h, zs_h,
                u_h, s_h,
                metav, idxv, dstv, asrows, xrows, adsl, accv, saccv, sslice,
                smem, sem, sem2):
    c_ = lax.axis_index("c")
    s_ = lax.axis_index("s")
    wid = s_ * 2 + c_
    pltpu.sync_copy(meta_h.at[wid], metav)
    mv = metav[...]
    n0 = mv[0]
    n1 = mv[1]
    e0al = mv[2]
    e1 = mv[3]
    n0 = pl.multiple_of(n0, 8)
    e0al = pl.multiple_of(e0al, 8)
    pltpu.sync_copy(ad_h.at[pl.ds(n0, _NRB)], adsl)
    pltpu.sync_copy(zrow_h, accv)
    pltpu.sync_copy(zs_h, sslice)
    saccv[...] = jnp.zeros((16,), jnp.float32)
    smem[0] = -1

    def flush(prevd):
        @pl.when((prevd >= n0) & (prevd < n1))
        def _():
            pltpu.sync_copy(accv, u_h.at[prevd])
            sslice[prevd - n0] = saccv[...]

    def chunk_body(ci, carry):
        base = pl.multiple_of(e0al + ci * _G, 8)

        @pl.when(base < e1)
        def _():
            pltpu.sync_copy(src_h.at[pl.ds(base, _G)], idxv)
            pltpu.sync_copy(dst_h.at[pl.ds(base, _G)], dstv.at[pl.ds(0, _G)])
            pltpu.async_copy(x_h.at[idxv], xrows, sem).wait()
            pltpu.async_copy(as_h.at[idxv], asrows, sem2).wait()

            def edge_body(j, carry2):
                d = dstv[pl.ds(j, 16)][0]
                prevd = smem[0]

                @pl.when(d != prevd)
                def _():
                    flush(prevd)
                    pltpu.sync_copy(zrow_h, accv)
                    saccv[...] = jnp.zeros((16,), jnp.float32)
                    smem[0] = d

                asv = asrows[j]
                adv = adsl[jnp.clip(d - n0, 0, _NRB - 1)]
                e16 = asv + adv
                e16 = jnp.where(e16 >= 0, e16, 0.2 * e16)
                ex = jnp.exp(e16)
                saccv[...] += ex
                ws = [ex[h] for h in range(H1)]
                for k in range(48):
                    rv = xrows[j, pl.ds(k * 16, 16)]
                    for h in range(H1):
                        plsc.addupdate(accv.at[h, pl.ds(k * 16, 16)],
                                       rv * ws[h])
                return carry2

            lax.fori_loop(0, _G, edge_body, 0, unroll=4)
        return carry

    lax.fori_loop(0, _NC_MAX, chunk_body, 0)
    flush(smem[0])
    pltpu.sync_copy(sslice.at[pl.ds(0, _NS)], s_h.at[pl.ds(n0, _NS)])

    @pl.when(wid == 31)
    def _():
        pltpu.sync_copy(sslice.at[pl.ds(_NS, 24)], s_h.at[pl.ds(9984, 24)])


def _edge1(x, as1, ad1p, srcp, dstp, meta, zrow, zs):
    f = pl.kernel(
        _edge1_body,
        out_type=[jax.ShapeDtypeStruct((N, H1, D), jnp.float32),
                  jax.ShapeDtypeStruct((NP, 16), jnp.float32)],
        mesh=_MESH,
        scratch_types=[pltpu.VMEM((16,), jnp.int32),
                       pltpu.VMEM((_G,), jnp.int32),
                       pltpu.VMEM((_G + 16,), jnp.int32),
                       pltpu.VMEM((_G, 16), jnp.float32),
                       pltpu.VMEM((_G, D), jnp.float32),
                       pltpu.VMEM((_NRB, 16), jnp.float32),
                       pltpu.VMEM((H1, D), jnp.float32),
                       pltpu.VMEM((16,), jnp.float32),
                       pltpu.VMEM((_NRB, 16), jnp.float32),
                       pltpu.SMEM((8,), jnp.int32),
                       pltpu.SemaphoreType.DMA,
                       pltpu.SemaphoreType.DMA],
        compiler_params=_SC_PARAMS,
    )
    return f(x, as1, ad1p, srcp, dstp, meta, zrow, zs)


def _l1mm_body(u1_ref, s1_ref, w1_ref, b1_ref, w2_ref, u2c_ref,
               xw2_ref, as2_ref, ad2_ref):
    h = pl.program_id(1)
    ub = u1_ref[...]
    oh = (lax.broadcasted_iota(jnp.int32, (16, 1), 0) == h).astype(jnp.float32)
    s_head = s1_ref[...] @ oh
    ohr = (lax.broadcasted_iota(jnp.int32, (1, 8), 1) == h).astype(jnp.float32)
    brow = ohr @ b1_ref[...]
    mm = ub @ w1_ref[...]
    t = jnp.where(s_head > 0, mm / (s_head + 1e-16), 0.0) + brow
    h1 = jnp.where(t > 0, t, jnp.exp(t) - 1.0)

    @pl.when(h == 0)
    def _():
        xw2_ref[...] = jnp.zeros_like(xw2_ref)
        as2_ref[...] = jnp.zeros_like(as2_ref)
        ad2_ref[...] = jnp.zeros_like(ad2_ref)

    xw2_ref[...] += h1 @ w2_ref[...]
    y = h1 @ u2c_ref[...]
    as2_ref[...] += y[:, :16]
    ad2_ref[...] += y[:, 16:32]


def _l1mm(u1, S1, W1, b1r, W2, U2cat):
    R = 2000
    return pl.pallas_call(
        _l1mm_body,
        grid=(5, H1),
        in_specs=[pl.BlockSpec((R, D), lambda r, h: (r, h)),
                  pl.BlockSpec((R, 16), lambda r, h: (r, 0)),
                  pl.BlockSpec((768, 768), lambda r, h: (0, h)),
                  pl.BlockSpec((8, 768), lambda r, h: (0, 0)),
                  pl.BlockSpec((768, 768), lambda r, h: (h, 0)),
                  pl.BlockSpec((768, 32), lambda r, h: (h, 0))],
        out_specs=[pl.BlockSpec((R, 768), lambda r, h: (r, 0)),
                   pl.BlockSpec((R, 16), lambda r, h: (r, 0)),
                   pl.BlockSpec((R, 16), lambda r, h: (r, 0))],
        out_shape=[jax.ShapeDtypeStruct((N, 768), jnp.float32),
                   jax.ShapeDtypeStruct((N, 16), jnp.float32),
                   jax.ShapeDtypeStruct((N, 16), jnp.float32)],
    )(u1, S1, W1, b1r, W2, U2cat)


def _edge2_body(xw2_h, as_h, ad_h, src_h, dst_h, meta_h, zrow_h, zs_h,
                u_h, s_h,
                metav, idxv, dstv, asrows, xrows, adsl, accv, saccv, sslice,
                smem, sem, sem2):
    c_ = lax.axis_index("c")
    s_ = lax.axis_index("s")
    wid = s_ * 2 + c_
    pltpu.sync_copy(meta_h.at[wid], metav)
    mv = metav[...]
    n0 = mv[0]
    n1 = mv[1]
    e0al = mv[2]
    e1 = mv[3]
    n0 = pl.multiple_of(n0, 8)
    e0al = pl.multiple_of(e0al, 8)
    pltpu.sync_copy(ad_h.at[pl.ds(n0, _NRB)], adsl)
    pltpu.sync_copy(zrow_h.at[0], accv)
    pltpu.sync_copy(zs_h, sslice)
    saccv[...] = jnp.zeros((16,), jnp.float32)
    smem[0] = -1
    smem[1] = 0

    def flush(prevd):
        @pl.when((prevd >= n0) & (prevd < n1) & (smem[1] > 0))
        def _():
            pltpu.sync_copy(accv, u_h.at[prevd])
            sslice[prevd - n0] = saccv[...]

    def chunk_body(ci, carry):
        base = pl.multiple_of(e0al + ci * _G, 8)

        @pl.when(base < e1)
        def _():
            pltpu.sync_copy(src_h.at[pl.ds(base, _G)], idxv)
            pltpu.sync_copy(dst_h.at[pl.ds(base, _G)], dstv.at[pl.ds(0, _G)])
            pltpu.async_copy(xw2_h.at[idxv], xrows, sem).wait()
            pltpu.async_copy(as_h.at[idxv], asrows, sem2).wait()

            def edge_body(j, carry2):
                d = dstv[pl.ds(j, 16)][0]
                prevd = smem[0]
                adv = adsl[jnp.clip(d - n0, 0, _NRB - 1)]
                mk = (adv[15] > 0.5).astype(jnp.int32)

                @pl.when(d != prevd)
                def _():
                    flush(prevd)
                    pltpu.sync_copy(zrow_h.at[0], accv)
                    saccv[...] = jnp.zeros((16,), jnp.float32)
                    smem[0] = d
                    smem[1] = mk

                @pl.when(mk > 0)
                def _():
                    asv = asrows[j]
                    e16 = asv + adv
                    e16 = jnp.where(e16 >= 0, e16, 0.2 * e16)
                    ex = jnp.exp(e16)
                    saccv[...] += ex
                    w0 = ex[0]
                    for k in range(48):
                        rv = xrows[j, pl.ds(k * 16, 16)]
                        plsc.addupdate(accv.at[pl.ds(k * 16, 16)], rv * w0)
                return carry2

            lax.fori_loop(0, _G, edge_body, 0, unroll=4)
        return carry

    lax.fori_loop(0, _NC_MAX, chunk_body, 0)
    flush(smem[0])
    pltpu.sync_copy(sslice.at[pl.ds(0, _NS)], s_h.at[pl.ds(n0, _NS)])

    @pl.when(wid == 31)
    def _():
        pltpu.sync_copy(sslice.at[pl.ds(_NS, 24)], s_h.at[pl.ds(9984, 24)])


def _edge2(xw2, as2, ad2p, src2p, dst2p, meta2, zrow, zs):
    f = pl.kernel(
        _edge2_body,
        out_type=[jax.ShapeDtypeStruct((NP, D), jnp.float32),
                  jax.ShapeDtypeStruct((NP, 16), jnp.float32)],
        mesh=_MESH,
        scratch_types=[pltpu.VMEM((16,), jnp.int32),
                       pltpu.VMEM((_G,), jnp.int32),
                       pltpu.VMEM((_G + 16,), jnp.int32),
                       pltpu.VMEM((_G, 16), jnp.float32),
                       pltpu.VMEM((_G, D), jnp.float32),
                       pltpu.VMEM((_NRB, 16), jnp.float32),
                       pltpu.VMEM((D,), jnp.float32),
                       pltpu.VMEM((16,), jnp.float32),
                       pltpu.VMEM((_NRB, 16), jnp.float32),
                       pltpu.SMEM((8,), jnp.int32),
                       pltpu.SemaphoreType.DMA,
                       pltpu.SemaphoreType.DMA],
        compiler_params=_SC_PARAMS,
    )
    return f(xw2, as2, ad2p, src2p, dst2p, meta2, zrow, zs)


def _gmask_body(u2_h, s2_h, x_h, mi_h, ug_h, sg_h, xe_h,
                miv, rows, srows, xrows, sem):
    c_ = lax.axis_index("c")
    s_ = lax.axis_index("s")
    wid = s_ * 2 + c_
    base = pl.multiple_of(wid * 32, 8)
    pltpu.sync_copy(mi_h.at[pl.ds(base, 32)], miv)
    pltpu.async_copy(u2_h.at[miv], rows, sem).wait()
    pltpu.sync_copy(rows, ug_h.at[pl.ds(base, 32)])
    pltpu.async_copy(s2_h.at[miv], srows, sem).wait()
    pltpu.sync_copy(srows, sg_h.at[pl.ds(base, 32)])
    pltpu.async_copy(x_h.at[miv], xrows, sem).wait()
    pltpu.sync_copy(xrows, xe_h.at[pl.ds(base, 32)])


def _gmask(u2, S2, x, mip):
    f = pl.kernel(
        _gmask_body,
        out_type=[jax.ShapeDtypeStruct((1024, D), jnp.float32),
                  jax.ShapeDtypeStruct((1024, 16), jnp.float32),
                  jax.ShapeDtypeStruct((1024, D), jnp.float32)],
        mesh=_MESH,
        scratch_types=[pltpu.VMEM((32,), jnp.int32),
                       pltpu.VMEM((32, D), jnp.float32),
                       pltpu.VMEM((32, 16), jnp.float32),
                       pltpu.VMEM((32, D), jnp.float32),
                       pltpu.SemaphoreType.DMA],
        compiler_params=_SC_PARAMS,
    )
    return f(u2, S2, x, mip)


def _final_body(ug_ref, sg_ref, xe_ref, wc_ref, bc_ref, b2_ref, out_ref):
    s = sg_ref[:, 0:1]
    g = jnp.where(s > 0, ug_ref[...] / (s + 1e-16), 0.0) + b2_ref[...]
    out_ref[...] = (g @ wc_ref[:768, :] + xe_ref[...] @ wc_ref[768:, :]
                    + bc_ref[...])


def _final(ug, sg, xe, Wcomb, bcomb, b2r):
    return pl.pallas_call(
        _final_body,
        out_shape=jax.ShapeDtypeStruct((1024, 2), jnp.float32),
    )(ug, sg, xe, Wcomb, bcomb, b2r)


def _meta_for(nstart, estart):
    n0 = nstart[:32]
    n1 = nstart[1:]
    e0 = estart[:32]
    e1 = estart[1:]
    e0al = e0 & ~jnp.int32(7)
    cols = [n0, n1, e0al, e1] + [jnp.zeros((32,), jnp.int32)] * 12
    return jnp.stack(cols, axis=1)


def kernel(x, edge_index, mask_idx, W1, a_src1, a_dst1, b1, W2, a_src2,
           a_dst2, b2, fc_W, fc_b, cls_W, cls_b):
    src = edge_index[0].astype(jnp.int32)
    dst = edge_index[1].astype(jnp.int32)
    mask_idx = mask_idx.astype(jnp.int32)

    packed = jnp.sort(dst * jnp.int32(16384) + src)
    dst_s = packed >> 14
    src_s = packed & jnp.int32(16383)
    tiles = jnp.arange(33, dtype=jnp.int32)
    nstart = jnp.minimum(tiles * _NS, N)
    nstart = jnp.where(tiles == 32, N, nstart)
    estart = jnp.searchsorted(dst_s, nstart).astype(jnp.int32)
    meta1 = _meta_for(nstart, estart)
    srcp = jnp.concatenate([src_s, jnp.zeros((64,), jnp.int32)])
    dstp = jnp.concatenate([dst_s, jnp.full((64,), N, jnp.int32)])
    marks = jnp.zeros((N,), jnp.float32).at[mask_idx].set(1.0)
    mip = jnp.concatenate([mask_idx, jnp.zeros((24,), jnp.int32)])

    zrow = jnp.zeros((H1, D), jnp.float32)
    zs = jnp.zeros((_NRB, 16), jnp.float32)

    Ucat1 = _prep1(W1, a_src1, a_dst1)
    U2cat, Wcomb, bcomb = _prep2(W2, a_src2, a_dst2, fc_W, cls_W, fc_b, cls_b)
    as1, ad1 = _asad1(x, Ucat1)
    ad1p = jnp.concatenate([ad1, jnp.zeros((NP - N, 16), jnp.float32)])
    u1, S1 = _edge1(x, as1, ad1p, srcp, dstp, meta1, zrow, zs)
    xw2, as2, ad2 = _l1mm(u1.reshape(N, H1 * D), S1[:N], W1,
                          b1.reshape(H1, D), W2, U2cat)
    ad2m = ad2.at[:, 15].set(marks)
    ad2p = jnp.concatenate([ad2m, jnp.zeros((NP - N, 16), jnp.float32)])
    u2, S2 = _edge2(xw2, as2, ad2p, srcp, dstp, meta1, zrow, zs)
    ug, sg, xe = _gmask(u2, S2, x, mip)
    outp = _final(ug, sg, xe, Wcomb, bcomb, b2[None, :])
    return outp[:1000]


# speedup vs baseline: 1.4739x; 1.4739x over previous
import functools

import jax
import jax.numpy as jnp
from jax import lax
from jax.experimental import pallas as pl
from jax.experimental.pallas import tpu as pltpu
from jax.experimental.pallas import tpu_sc as plsc

N = 10000
NP = 10016
E = 64000
EP = E + 64
D = 768
H1 = 8

_G = 96
_NC_MAX = (E + _G - 1) // _G + 1
_NS = 312
_NRB = 336

_SC_PARAMS = pltpu.CompilerParams(use_tc_tiling_on_sc=False)
_MESH = plsc.VectorSubcoreMesh(core_axis_name="c", subcore_axis_name="s")


def _lane_bcast(vec, lane):
    idx = jnp.full((16, 1), lane, jnp.int32)
    return lax.gather(
        vec, idx,
        lax.GatherDimensionNumbers(offset_dims=(), collapsed_slice_dims=(0,),
                                   start_index_map=(0,)),
        slice_sizes=(1,), mode=lax.GatherScatterMode.PROMISE_IN_BOUNDS)


def _prep1_body(w1_ref, as_ref, ad_ref, out_ref):
    h = pl.program_id(0)
    ohr = (lax.broadcasted_iota(jnp.int32, (1, 8), 1) == h).astype(jnp.float32)
    a_s = ohr @ as_ref[...]
    a_d = ohr @ ad_ref[...]
    us = lax.dot_general(w1_ref[...], a_s, (((1,), (1,)), ((), ())))
    ud = lax.dot_general(w1_ref[...], a_d, (((1,), (1,)), ((), ())))
    row = lax.broadcasted_iota(jnp.int32, (1, 32), 1)
    ohs = (row == h).astype(jnp.float32)
    ohd = (row == (h + 16)).astype(jnp.float32)

    @pl.when(h == 0)
    def _():
        out_ref[...] = jnp.zeros_like(out_ref)

    out_ref[...] += us @ ohs + ud @ ohd


def _prep1(W1, a1s, a1d):
    return pl.pallas_call(
        _prep1_body,
        grid=(H1,),
        in_specs=[pl.BlockSpec((768, 768), lambda h: (0, h)),
                  pl.BlockSpec((8, 768), lambda h: (0, 0)),
                  pl.BlockSpec((8, 768), lambda h: (0, 0))],
        out_specs=pl.BlockSpec((768, 32), lambda h: (0, 0)),
        out_shape=jax.ShapeDtypeStruct((768, 32), jnp.float32),
    )(W1, a1s, a1d)


def _prep2_body(w2_ref, a2s_ref, a2d_ref, fcw_ref, clsw_ref, fcb_ref, clsb_ref,
                u2c_ref, wc_ref, bc_ref):
    u2s = lax.dot_general(w2_ref[...], a2s_ref[...], (((1,), (1,)), ((), ())))
    u2d = lax.dot_general(w2_ref[...], a2d_ref[...], (((1,), (1,)), ((), ())))
    row = lax.broadcasted_iota(jnp.int32, (1, 32), 1)
    oh0 = (row == 0).astype(jnp.float32)
    oh16 = (row == 16).astype(jnp.float32)
    u2c_ref[...] = u2s @ oh0 + u2d @ oh16
    wc = fcw_ref[...] @ clsw_ref[...]
    wc_ref[...] = wc
    bc_ref[...] = fcb_ref[...] @ clsw_ref[...] + clsb_ref[...]


def _prep2(W2, a2s, a2d, fc_W, cls_W, fc_b, cls_b):
    return pl.pallas_call(
        _prep2_body,
        out_shape=[jax.ShapeDtypeStruct((6144, 32), jnp.float32),
                   jax.ShapeDtypeStruct((1536, 2), jnp.float32),
                   jax.ShapeDtypeStruct((1, 2), jnp.float32)],
    )(W2, a2s, a2d, fc_W, cls_W, fc_b[None, :], cls_b[None, :])


def _asad1_body(x_ref, u_ref, as_ref, ad_ref):
    y = x_ref[...] @ u_ref[...]
    as_ref[...] = y[:, :16]
    ad_ref[...] = y[:, 16:32]


def _asad1(x, Ucat1):
    return pl.pallas_call(
        _asad1_body,
        grid=(5,),
        in_specs=[pl.BlockSpec((2000, 768), lambda r: (r, 0)),
                  pl.BlockSpec((768, 32), lambda r: (0, 0))],
        out_specs=[pl.BlockSpec((2000, 16), lambda r: (r, 0)),
                   pl.BlockSpec((2000, 16), lambda r: (r, 0))],
        out_shape=[jax.ShapeDtypeStruct((N, 16), jnp.float32),
                   jax.ShapeDtypeStruct((N, 16), jnp.float32)],
    )(x, Ucat1)


def _edge1_body(x_h, as_h, ad_h, src_h, dst_h, meta_h, zrow_h, zs_h,
                u_h, s_h,
                metav, idxv, dstv, asrows, xrows, adsl, accv, saccv, sslice,
                smem, sem, sem2):
    c_ = lax.axis_index("c")
    s_ = lax.axis_index("s")
    wid = s_ * 2 + c_
    pltpu.sync_copy(meta_h.at[wid], metav)
    mv = metav[...]
    n0 = mv[0]
    n1 = mv[1]
    e0al = mv[2]
    e1 = mv[3]
    n0 = pl.multiple_of(n0, 8)
    e0al = pl.multiple_of(e0al, 8)
    pltpu.sync_copy(ad_h.at[pl.ds(n0, _NRB)], adsl)
    pltpu.sync_copy(zrow_h, accv)
    pltpu.sync_copy(zs_h, sslice)
    saccv[...] = jnp.zeros((16,), jnp.float32)
    smem[0] = -1

    def flush(prevd):
        @pl.when((prevd >= n0) & (prevd < n1))
        def _():
            pltpu.sync_copy(accv, u_h.at[prevd])
            sslice[prevd - n0] = saccv[...]

    def chunk_body(ci, carry):
        base = pl.multiple_of(e0al + ci * _G, 8)

        @pl.when(base < e1)
        def _():
            pltpu.sync_copy(src_h.at[pl.ds(base, _G)], idxv)
            pltpu.sync_copy(dst_h.at[pl.ds(base, _G)], dstv.at[pl.ds(0, _G)])
            pltpu.async_copy(x_h.at[idxv], xrows, sem).wait()
            pltpu.async_copy(as_h.at[idxv], asrows, sem2).wait()

            def edge_body(j, carry2):
                d = dstv[pl.ds(j, 16)][0]
                prevd = smem[0]

                @pl.when(d != prevd)
                def _():
                    flush(prevd)
                    pltpu.sync_copy(zrow_h, accv)
                    saccv[...] = jnp.zeros((16,), jnp.float32)
                    smem[0] = d

                asv = asrows[j]
                adv = adsl[jnp.clip(d - n0, 0, _NRB - 1)]
                e16 = asv + adv
                e16 = jnp.where(e16 >= 0, e16, 0.2 * e16)
                ex = jnp.exp(e16)
                saccv[...] += ex
                ws = [ex[h] for h in range(H1)]
                for k in range(48):
                    rv = xrows[j, pl.ds(k * 16, 16)]
                    for h in range(H1):
                        plsc.addupdate(accv.at[h, pl.ds(k * 16, 16)],
                                       rv * ws[h])
                return carry2

            lax.fori_loop(0, _G, edge_body, 0)
        return carry

    lax.fori_loop(0, _NC_MAX, chunk_body, 0)
    flush(smem[0])
    pltpu.sync_copy(sslice.at[pl.ds(0, _NS)], s_h.at[pl.ds(n0, _NS)])

    @pl.when(wid == 31)
    def _():
        pltpu.sync_copy(sslice.at[pl.ds(_NS, 24)], s_h.at[pl.ds(9984, 24)])


def _edge1(x, as1, ad1p, srcp, dstp, meta, zrow, zs):
    f = pl.kernel(
        _edge1_body,
        out_type=[jax.ShapeDtypeStruct((N, H1, D), jnp.float32),
                  jax.ShapeDtypeStruct((NP, 16), jnp.float32)],
        mesh=_MESH,
        scratch_types=[pltpu.VMEM((16,), jnp.int32),
                       pltpu.VMEM((_G,), jnp.int32),
                       pltpu.VMEM((_G + 16,), jnp.int32),
                       pltpu.VMEM((_G, 16), jnp.float32),
                       pltpu.VMEM((_G, D), jnp.float32),
                       pltpu.VMEM((_NRB, 16), jnp.float32),
                       pltpu.VMEM((H1, D), jnp.float32),
                       pltpu.VMEM((16,), jnp.float32),
                       pltpu.VMEM((_NRB, 16), jnp.float32),
                       pltpu.SMEM((8,), jnp.int32),
                       pltpu.SemaphoreType.DMA,
                       pltpu.SemaphoreType.DMA],
        compiler_params=_SC_PARAMS,
    )
    return f(x, as1, ad1p, srcp, dstp, meta, zrow, zs)


def _l1mm_body(u1_ref, s1_ref, w1_ref, b1_ref, w2_ref, u2c_ref,
               xw2_ref, as2_ref, ad2_ref):
    h = pl.program_id(1)
    ub = u1_ref[...]
    oh = (lax.broadcasted_iota(jnp.int32, (16, 1), 0) == h).astype(jnp.float32)
    s_head = s1_ref[...] @ oh
    ohr = (lax.broadcasted_iota(jnp.int32, (1, 8), 1) == h).astype(jnp.float32)
    brow = ohr @ b1_ref[...]
    mm = ub @ w1_ref[...]
    t = jnp.where(s_head > 0, mm / (s_head + 1e-16), 0.0) + brow
    h1 = jnp.where(t > 0, t, jnp.exp(t) - 1.0)

    @pl.when(h == 0)
    def _():
        xw2_ref[...] = jnp.zeros_like(xw2_ref)
        as2_ref[...] = jnp.zeros_like(as2_ref)
        ad2_ref[...] = jnp.zeros_like(ad2_ref)

    xw2_ref[...] += h1 @ w2_ref[...]
    y = h1 @ u2c_ref[...]
    as2_ref[...] += y[:, :16]
    ad2_ref[...] += y[:, 16:32]


def _l1mm(u1, S1, W1, b1r, W2, U2cat):
    R = 2000
    return pl.pallas_call(
        _l1mm_body,
        grid=(5, H1),
        in_specs=[pl.BlockSpec((R, D), lambda r, h: (r, h)),
                  pl.BlockSpec((R, 16), lambda r, h: (r, 0)),
                  pl.BlockSpec((768, 768), lambda r, h: (0, h)),
                  pl.BlockSpec((8, 768), lambda r, h: (0, 0)),
                  pl.BlockSpec((768, 768), lambda r, h: (h, 0)),
                  pl.BlockSpec((768, 32), lambda r, h: (h, 0))],
        out_specs=[pl.BlockSpec((R, 768), lambda r, h: (r, 0)),
                   pl.BlockSpec((R, 16), lambda r, h: (r, 0)),
                   pl.BlockSpec((R, 16), lambda r, h: (r, 0))],
        out_shape=[jax.ShapeDtypeStruct((N, 768), jnp.float32),
                   jax.ShapeDtypeStruct((N, 16), jnp.float32),
                   jax.ShapeDtypeStruct((N, 16), jnp.float32)],
    )(u1, S1, W1, b1r, W2, U2cat)


def _edge2_body(xw2_h, as_h, ad_h, src_h, dst_h, meta_h, zrow_h, zs_h,
                u_h, s_h,
                metav, idxv, dstv, asrows, xrows, adsl, accv, saccv, sslice,
                smem, sem, sem2):
    c_ = lax.axis_index("c")
    s_ = lax.axis_index("s")
    wid = s_ * 2 + c_
    pltpu.sync_copy(meta_h.at[wid], metav)
    mv = metav[...]
    n0 = mv[0]
    n1 = mv[1]
    e0al = mv[2]
    e1 = mv[3]
    n0 = pl.multiple_of(n0, 8)
    e0al = pl.multiple_of(e0al, 8)
    pltpu.sync_copy(ad_h.at[pl.ds(n0, _NRB)], adsl)
    pltpu.sync_copy(zrow_h.at[0], accv)
    pltpu.sync_copy(zs_h, sslice)
    saccv[...] = jnp.zeros((16,), jnp.float32)
    smem[0] = -1
    smem[1] = 0

    def flush(prevd):
        @pl.when((prevd >= n0) & (prevd < n1) & (smem[1] > 0))
        def _():
            pltpu.sync_copy(accv, u_h.at[prevd])
            sslice[prevd - n0] = saccv[...]

    def chunk_body(ci, carry):
        base = pl.multiple_of(e0al + ci * _G, 8)

        @pl.when(base < e1)
        def _():
            pltpu.sync_copy(src_h.at[pl.ds(base, _G)], idxv)
            pltpu.sync_copy(dst_h.at[pl.ds(base, _G)], dstv.at[pl.ds(0, _G)])
            pltpu.async_copy(xw2_h.at[idxv], xrows, sem).wait()
            pltpu.async_copy(as_h.at[idxv], asrows, sem2).wait()

            def edge_body(j, carry2):
                d = dstv[pl.ds(j, 16)][0]
                prevd = smem[0]
                adv = adsl[jnp.clip(d - n0, 0, _NRB - 1)]
                mk = (adv[15] > 0.5).astype(jnp.int32)

                @pl.when(d != prevd)
                def _():
                    flush(prevd)
                    pltpu.sync_copy(zrow_h.at[0], accv)
                    saccv[...] = jnp.zeros((16,), jnp.float32)
                    smem[0] = d
                    smem[1] = mk

                @pl.when(mk > 0)
                def _():
                    asv = asrows[j]
                    e16 = asv + adv
                    e16 = jnp.where(e16 >= 0, e16, 0.2 * e16)
                    ex = jnp.exp(e16)
                    saccv[...] += ex
                    w0 = ex[0]
                    for k in range(48):
                        rv = xrows[j, pl.ds(k * 16, 16)]
                        plsc.addupdate(accv.at[pl.ds(k * 16, 16)], rv * w0)
                return carry2

            lax.fori_loop(0, _G, edge_body, 0)
        return carry

    lax.fori_loop(0, _NC_MAX, chunk_body, 0)
    flush(smem[0])
    pltpu.sync_copy(sslice.at[pl.ds(0, _NS)], s_h.at[pl.ds(n0, _NS)])

    @pl.when(wid == 31)
    def _():
        pltpu.sync_copy(sslice.at[pl.ds(_NS, 24)], s_h.at[pl.ds(9984, 24)])


def _edge2(xw2, as2, ad2p, src2p, dst2p, meta2, zrow, zs):
    f = pl.kernel(
        _edge2_body,
        out_type=[jax.ShapeDtypeStruct((NP, D), jnp.float32),
                  jax.ShapeDtypeStruct((NP, 16), jnp.float32)],
        mesh=_MESH,
        scratch_types=[pltpu.VMEM((16,), jnp.int32),
                       pltpu.VMEM((_G,), jnp.int32),
                       pltpu.VMEM((_G + 16,), jnp.int32),
                       pltpu.VMEM((_G, 16), jnp.float32),
                       pltpu.VMEM((_G, D), jnp.float32),
                       pltpu.VMEM((_NRB, 16), jnp.float32),
                       pltpu.VMEM((D,), jnp.float32),
                       pltpu.VMEM((16,), jnp.float32),
                       pltpu.VMEM((_NRB, 16), jnp.float32),
                       pltpu.SMEM((8,), jnp.int32),
                       pltpu.SemaphoreType.DMA,
                       pltpu.SemaphoreType.DMA],
        compiler_params=_SC_PARAMS,
    )
    return f(xw2, as2, ad2p, src2p, dst2p, meta2, zrow, zs)


def _gmask_body(u2_h, s2_h, x_h, mi_h, ug_h, sg_h, xe_h,
                miv, rows, srows, xrows, sem):
    c_ = lax.axis_index("c")
    s_ = lax.axis_index("s")
    wid = s_ * 2 + c_
    base = pl.multiple_of(wid * 32, 8)
    pltpu.sync_copy(mi_h.at[pl.ds(base, 32)], miv)
    pltpu.async_copy(u2_h.at[miv], rows, sem).wait()
    pltpu.sync_copy(rows, ug_h.at[pl.ds(base, 32)])
    pltpu.async_copy(s2_h.at[miv], srows, sem).wait()
    pltpu.sync_copy(srows, sg_h.at[pl.ds(base, 32)])
    pltpu.async_copy(x_h.at[miv], xrows, sem).wait()
    pltpu.sync_copy(xrows, xe_h.at[pl.ds(base, 32)])


def _gmask(u2, S2, x, mip):
    f = pl.kernel(
        _gmask_body,
        out_type=[jax.ShapeDtypeStruct((1024, D), jnp.float32),
                  jax.ShapeDtypeStruct((1024, 16), jnp.float32),
                  jax.ShapeDtypeStruct((1024, D), jnp.float32)],
        mesh=_MESH,
        scratch_types=[pltpu.VMEM((32,), jnp.int32),
                       pltpu.VMEM((32, D), jnp.float32),
                       pltpu.VMEM((32, 16), jnp.float32),
                       pltpu.VMEM((32, D), jnp.float32),
                       pltpu.SemaphoreType.DMA],
        compiler_params=_SC_PARAMS,
    )
    return f(u2, S2, x, mip)


def _final_body(ug_ref, sg_ref, xe_ref, wc_ref, bc_ref, b2_ref, out_ref):
    s = sg_ref[:, 0:1]
    g = jnp.where(s > 0, ug_ref[...] / (s + 1e-16), 0.0) + b2_ref[...]
    out_ref[...] = (g @ wc_ref[:768, :] + xe_ref[...] @ wc_ref[768:, :]
                    + bc_ref[...])


def _final(ug, sg, xe, Wcomb, bcomb, b2r):
    return pl.pallas_call(
        _final_body,
        out_shape=jax.ShapeDtypeStruct((1024, 2), jnp.float32),
    )(ug, sg, xe, Wcomb, bcomb, b2r)


def _meta_for(nstart, estart):
    n0 = nstart[:32]
    n1 = nstart[1:]
    e0 = estart[:32]
    e1 = estart[1:]
    e0al = e0 & ~jnp.int32(7)
    cols = [n0, n1, e0al, e1] + [jnp.zeros((32,), jnp.int32)] * 12
    return jnp.stack(cols, axis=1)


def kernel(x, edge_index, mask_idx, W1, a_src1, a_dst1, b1, W2, a_src2,
           a_dst2, b2, fc_W, fc_b, cls_W, cls_b):
    src = edge_index[0].astype(jnp.int32)
    dst = edge_index[1].astype(jnp.int32)
    mask_idx = mask_idx.astype(jnp.int32)

    packed = jnp.sort(dst * jnp.int32(16384) + src)
    dst_s = packed >> 14
    src_s = packed & jnp.int32(16383)
    tiles = jnp.arange(33, dtype=jnp.int32)
    nstart = jnp.minimum(tiles * _NS, N)
    nstart = jnp.where(tiles == 32, N, nstart)
    estart = jnp.searchsorted(dst_s, nstart).astype(jnp.int32)
    meta1 = _meta_for(nstart, estart)
    srcp = jnp.concatenate([src_s, jnp.zeros((64,), jnp.int32)])
    dstp = jnp.concatenate([dst_s, jnp.full((64,), N, jnp.int32)])
    marks = jnp.zeros((N,), jnp.float32).at[mask_idx].set(1.0)
    mip = jnp.concatenate([mask_idx, jnp.zeros((24,), jnp.int32)])

    zrow = jnp.zeros((H1, D), jnp.float32)
    zs = jnp.zeros((_NRB, 16), jnp.float32)

    Ucat1 = _prep1(W1, a_src1, a_dst1)
    U2cat, Wcomb, bcomb = _prep2(W2, a_src2, a_dst2, fc_W, cls_W, fc_b, cls_b)
    as1, ad1 = _asad1(x, Ucat1)
    ad1p = jnp.concatenate([ad1, jnp.zeros((NP - N, 16), jnp.float32)])
    u1, S1 = _edge1(x, as1, ad1p, srcp, dstp, meta1, zrow, zs)
    xw2, as2, ad2 = _l1mm(u1.reshape(N, H1 * D), S1[:N], W1,
                          b1.reshape(H1, D), W2, U2cat)
    ad2m = ad2.at[:, 15].set(marks)
    ad2p = jnp.concatenate([ad2m, jnp.zeros((NP - N, 16), jnp.float32)])
    u2, S2 = _edge2(xw2, as2, ad2p, srcp, dstp, meta1, zrow, zs)
    ug, sg, xe = _gmask(u2, S2, x, mip)
    outp = _final(ug, sg, xe, Wcomb, bcomb, b2[None, :])
    return outp[:1000]
